# baseline (speedup 1.0000x reference)
import jax
import jax.numpy as jnp
from jax import lax
from jax.experimental import pallas as pl
from jax.experimental.pallas import tpu as pltpu
from jax.experimental.pallas import tpu_sc as plsc

_T = 16384
_E = 64
_NC, _NS, _L = 2, 16, 16
_NW = _NC * _NS
_RPW = _T // _NW
_H = _RPW // 2
_GH = _H // _L
_EB = _E // _L


def _sc_body(qt_hbm, out_hbm, chunk, m_ref, a_ref, obuf, sem0, sem1):
    wid = lax.axis_index("s") * _NC + lax.axis_index("c")
    base = wid * _RPW
    cp0 = pltpu.async_copy(
        qt_hbm.at[:, pl.ds(base, _H)], chunk.at[:, pl.ds(0, _H)], sem0)
    cp1 = pltpu.async_copy(
        qt_hbm.at[:, pl.ds(base + _H, _H)], chunk.at[:, pl.ds(_H, _H)], sem1)

    zf = jnp.zeros((_L,), jnp.float32)
    ones = jnp.ones((_L,), jnp.float32)
    lane = lax.iota(jnp.int32, _L)

    for half in range(2):
        (cp0 if half == 0 else cp1).wait()
        for eb in range(_EB):
            def g_body(g, accs, eb=eb):
                accs = list(accs)
                v = [chunk[eb * _L + j, pl.ds(g * _L, _L)] for j in range(_L)]
                for j in range(_L):
                    accs[j] = accs[j] + v[j]
                mt = list(v)
                at = [jnp.full((_L,), eb * _L + j, jnp.int32)
                      for j in range(_L)]
                n = 1
                while n < _L:
                    for j in range(0, _L, 2 * n):
                        ge = mt[j] >= mt[j + n]
                        mt[j] = jnp.where(ge, mt[j], mt[j + n])
                        at[j] = jnp.where(ge, at[j], at[j + n])
                    n *= 2
                if eb == 0:
                    m, a = mt[0], at[0]
                else:
                    gt = mt[0] > m_ref[g]
                    m = jnp.where(gt, mt[0], m_ref[g])
                    a = jnp.where(gt, at[0], a_ref[g])
                m_ref[g] = m
                a_ref[g] = a
                return tuple(accs)

            accs = lax.fori_loop(
                0, _GH, lambda g, c, f=g_body, h=half: f(g + h * _GH, c),
                (zf,) * _L)
            ovec = zf
            for j in range(_L):
                ovec = jnp.where(lane == j, jnp.sum(accs[j]), ovec)
            if half == 0:
                obuf[pl.ds(eb * _L, _L)] = ovec
            else:
                obuf[pl.ds(eb * _L, _L)] = obuf[pl.ds(eb * _L, _L)] + ovec

    for j in range(_EB):
        obuf[pl.ds(_E + j * _L, _L)] = zf

    def h_body(g, carry):
        plsc.addupdate_scatter(obuf.at[pl.ds(_E, _E)], [a_ref[g]], ones)
        return carry

    lax.fori_loop(0, 2 * _GH, h_body, 0)
    pltpu.sync_copy(obuf, out_hbm.at[wid])


_sc_call = pl.kernel(
    _sc_body,
    out_type=jax.ShapeDtypeStruct((_NW, 2 * _E), jnp.float32),
    mesh=plsc.VectorSubcoreMesh(core_axis_name="c", subcore_axis_name="s"),
    compiler_params=pltpu.CompilerParams(needs_layout_passes=False),
    scratch_types=[
        pltpu.VMEM((_E, _RPW), jnp.float32),
        pltpu.VMEM((2 * _GH, _L), jnp.float32),
        pltpu.VMEM((2 * _GH, _L), jnp.int32),
        pltpu.VMEM((2 * _E,), jnp.float32),
        pltpu.SemaphoreType.DMA,
        pltpu.SemaphoreType.DMA,
    ],
)


def _tc_body(p_ref, o_ref):
    s = jnp.sum(p_ref[...], axis=0)
    o_ref[...] = (jnp.sum(s[:_E] * s[_E:]) * (_E / (_T * _T))).reshape(1, 1)


def kernel(q):
    parts = _sc_call(q.T)
    out = pl.pallas_call(
        _tc_body,
        out_shape=jax.ShapeDtypeStruct((1, 1), jnp.float32),
    )(parts)
    return out[0, 0]


# speedup vs baseline: 1.0134x; 1.0134x over previous
import jax
import jax.numpy as jnp
from jax import lax
from jax.experimental import pallas as pl
from jax.experimental.pallas import tpu as pltpu
from jax.experimental.pallas import tpu_sc as plsc

_T = 16384
_E = 64
_NC, _NS, _L = 2, 16, 16
_NW = _NC * _NS
_RPW = _T // _NW
_H = _RPW // 2
_GH = _H // _L
_EB = _E // _L
_CSB = 8


def _sc_body(qt_hbm, hist_hbm, chunk, m_ref, a_ref, obuf, sem0, sem1):
    wid = lax.axis_index("s") * _NC + lax.axis_index("c")
    base = wid * _RPW
    cp0 = pltpu.async_copy(
        qt_hbm.at[:, pl.ds(base, _H)], chunk.at[:, pl.ds(0, _H)], sem0)
    cp1 = pltpu.async_copy(
        qt_hbm.at[:, pl.ds(base + _H, _H)], chunk.at[:, pl.ds(_H, _H)], sem1)

    zf = jnp.zeros((_L,), jnp.float32)
    ones = jnp.ones((_L,), jnp.float32)

    for half in range(2):
        (cp0 if half == 0 else cp1).wait()
        for eb in range(_EB):
            def g_body(g, carry, eb=eb):
                v = [chunk[eb * _L + j, pl.ds(g * _L, _L)] for j in range(_L)]
                mt = list(v)
                at = [jnp.full((_L,), eb * _L + j, jnp.int32)
                      for j in range(_L)]
                n = 1
                while n < _L:
                    for j in range(0, _L, 2 * n):
                        ge = mt[j] >= mt[j + n]
                        mt[j] = jnp.where(ge, mt[j], mt[j + n])
                        at[j] = jnp.where(ge, at[j], at[j + n])
                    n *= 2
                if eb == 0:
                    m, a = mt[0], at[0]
                else:
                    gt = mt[0] > m_ref[g]
                    m = jnp.where(gt, mt[0], m_ref[g])
                    a = jnp.where(gt, at[0], a_ref[g])
                m_ref[g] = m
                a_ref[g] = a
                return carry

            lax.fori_loop(
                0, _GH, lambda g, c, f=g_body, h=half: f(g + h * _GH, c), 0)

    for j in range(_EB):
        obuf[pl.ds(j * _L, _L)] = zf

    def h_body(g, carry):
        plsc.addupdate_scatter(obuf, [a_ref[g]], ones)
        return carry

    lax.fori_loop(0, 2 * _GH, h_body, 0)
    pltpu.sync_copy(obuf, hist_hbm.at[wid])


_sc_call = pl.kernel(
    _sc_body,
    out_type=jax.ShapeDtypeStruct((_NW, _E), jnp.float32),
    mesh=plsc.VectorSubcoreMesh(core_axis_name="c", subcore_axis_name="s"),
    compiler_params=pltpu.CompilerParams(needs_layout_passes=False),
    scratch_types=[
        pltpu.VMEM((_E, _RPW), jnp.float32),
        pltpu.VMEM((2 * _GH, _L), jnp.float32),
        pltpu.VMEM((2 * _GH, _L), jnp.int32),
        pltpu.VMEM((_E,), jnp.float32),
        pltpu.SemaphoreType.DMA,
        pltpu.SemaphoreType.DMA,
    ],
)


def _cs_body(qt_ref, cs_ref):
    b = pl.program_id(0)
    s = jnp.sum(qt_ref[...], axis=1).reshape(1, _E)

    @pl.when(b == 0)
    def _init():
        cs_ref[...] = s

    @pl.when(b != 0)
    def _acc():
        cs_ref[...] = cs_ref[...] + s


def _fin_body(cs_ref, h_ref, o_ref):
    ct = jnp.sum(h_ref[...], axis=0)
    o_ref[...] = (jnp.sum(cs_ref[0, :] * ct) * (_E / (_T * _T))).reshape(1, 1)


def kernel(q):
    qt = q.T
    hist = _sc_call(qt)
    cs = pl.pallas_call(
        _cs_body,
        grid=(_CSB,),
        in_specs=[pl.BlockSpec((_E, _T // _CSB), lambda b: (0, b))],
        out_specs=pl.BlockSpec((1, _E), lambda b: (0, 0)),
        out_shape=jax.ShapeDtypeStruct((1, _E), jnp.float32),
    )(qt)
    out = pl.pallas_call(
        _fin_body,
        out_shape=jax.ShapeDtypeStruct((1, 1), jnp.float32),
    )(cs, hist)
    return out[0, 0]


# speedup vs baseline: 1.0769x; 1.0626x over previous
import jax
import jax.numpy as jnp
from jax import lax
from jax.experimental import pallas as pl
from jax.experimental.pallas import tpu as pltpu
from jax.experimental.pallas import tpu_sc as plsc

_T = 16384
_E = 64
_NC, _NS, _L = 2, 16, 16
_NW = _NC * _NS
_RPW = _T // _NW
_H = _RPW // 2
_GH = _H // _L
_CSB = 8


def _sc_body(qt_hbm, hist_hbm, chunk, obuf, sem0, sem1):
    wid = lax.axis_index("s") * _NC + lax.axis_index("c")
    base = wid * _RPW
    cp0 = pltpu.async_copy(
        qt_hbm.at[:, pl.ds(base, _H)], chunk.at[:, pl.ds(0, _H)], sem0)
    cp1 = pltpu.async_copy(
        qt_hbm.at[:, pl.ds(base + _H, _H)], chunk.at[:, pl.ds(_H, _H)], sem1)

    zf = jnp.zeros((_L,), jnp.float32)
    ones = jnp.ones((_L,), jnp.float32)
    for j in range(_E // _L):
        obuf[pl.ds(j * _L, _L)] = zf

    def g_body(g, carry):
        v = [chunk[e, pl.ds(g * _L, _L)] for e in range(_E)]
        mt = list(v)
        at = [jnp.full((_L,), e, jnp.int32) for e in range(_E)]
        n = 1
        while n < _E:
            for j in range(0, _E, 2 * n):
                ge = mt[j] >= mt[j + n]
                mt[j] = jnp.where(ge, mt[j], mt[j + n])
                at[j] = jnp.where(ge, at[j], at[j + n])
            n *= 2
        plsc.addupdate_scatter(obuf, [at[0]], ones)
        return carry

    cp0.wait()
    lax.fori_loop(0, _GH, g_body, 0)
    cp1.wait()
    lax.fori_loop(_GH, 2 * _GH, g_body, 0)

    pltpu.sync_copy(obuf, hist_hbm.at[wid])


_sc_call = pl.kernel(
    _sc_body,
    out_type=jax.ShapeDtypeStruct((_NW, _E), jnp.float32),
    mesh=plsc.VectorSubcoreMesh(core_axis_name="c", subcore_axis_name="s"),
    compiler_params=pltpu.CompilerParams(needs_layout_passes=False),
    scratch_types=[
        pltpu.VMEM((_E, _RPW), jnp.float32),
        pltpu.VMEM((_E,), jnp.float32),
        pltpu.SemaphoreType.DMA,
        pltpu.SemaphoreType.DMA,
    ],
)


def _cs_body(qt_ref, cs_ref, acc_ref):
    b = pl.program_id(0)
    v = qt_ref[...].reshape(_E, (_T // _CSB) // 128, 128)
    s = jnp.sum(v, axis=1)

    @pl.when(b == 0)
    def _init():
        acc_ref[...] = s

    @pl.when(b != 0)
    def _acc():
        acc_ref[...] = acc_ref[...] + s

    @pl.when(b == _CSB - 1)
    def _fin():
        cs_ref[...] = jnp.sum(acc_ref[...], axis=1).reshape(1, _E)


def _fin_body(cs_ref, h_ref, o_ref):
    ct = jnp.sum(h_ref[...], axis=0)
    o_ref[...] = (jnp.sum(cs_ref[0, :] * ct) * (_E / (_T * _T))).reshape(1, 1)


def kernel(q):
    qt = q.T
    hist = _sc_call(qt)
    cs = pl.pallas_call(
        _cs_body,
        grid=(_CSB,),
        in_specs=[pl.BlockSpec((_E, _T // _CSB), lambda b: (0, b))],
        out_specs=pl.BlockSpec((1, _E), lambda b: (0, 0)),
        out_shape=jax.ShapeDtypeStruct((1, _E), jnp.float32),
        scratch_shapes=[pltpu.VMEM((_E, 128), jnp.float32)],
    )(qt)
    out = pl.pallas_call(
        _fin_body,
        out_shape=jax.ShapeDtypeStruct((1, 1), jnp.float32),
    )(cs, hist)
    return out[0, 0]
